# baseline (speedup 1.0000x reference)
import jax
import jax.numpy as jnp
from jax import lax
from jax.experimental import pallas as pl
from jax.experimental.pallas import tpu as pltpu
from jax.experimental.pallas import tpu_sc as plsc

_NW = 32
_CHUNK = 512
_TLOW = 32
_BIG = 1.0e9


def _sc_first_spike(T, BN):
    nchunk_total = BN // _CHUNK
    rounds = -(-nchunk_total // _NW)
    thigh = T - _TLOW
    ngrp = _CHUNK // 16
    mesh = plsc.VectorSubcoreMesh(core_axis_name="c", subcore_axis_name="s")

    def body(x_hbm, out_hbm, lowbuf, highbuf, outbuf, flagbuf, sem):
        cid = lax.axis_index("c")
        sid = lax.axis_index("s")
        wid = sid * 2 + cid

        def round_body(k, carry):
            chunk_id = k * _NW + wid

            @pl.when(chunk_id < nchunk_total)
            def _do_chunk():
                _chunk(chunk_id)
            return carry

        def _chunk(chunk_id):
            c0 = pl.multiple_of(chunk_id * _CHUNK, 128)
            pltpu.async_copy(
                x_hbm.at[pl.ds(0, _TLOW), pl.ds(c0, _CHUNK)], lowbuf, sem
            ).wait()

            def grp_body(g, silent_vec):
                goff = pl.multiple_of(g * 16, 16)
                acc = jnp.full((16,), _BIG, jnp.float32)
                for t in range(_TLOW - 1, -1, -1):
                    x = lowbuf[t, pl.ds(goff, 16)]
                    acc = jnp.where(x > 0.5, jnp.float32(t), acc)
                outbuf[pl.ds(goff, 16)] = acc
                return silent_vec | jnp.where(acc >= _BIG, 1, 0)

            silent_vec = lax.fori_loop(
                0, ngrp, grp_body, jnp.zeros((16,), jnp.int32)
            )
            total = silent_vec[0]
            for lane in range(1, 16):
                total = total | silent_vec[lane]

            @pl.when(total > 0)
            def _rare():
                pltpu.async_copy(
                    x_hbm.at[pl.ds(_TLOW, thigh), pl.ds(c0, _CHUNK)],
                    highbuf,
                    sem,
                ).wait()

                def grp2(g, carry2):
                    goff = pl.multiple_of(g * 16, 16)
                    acc2 = jnp.full((16,), jnp.float32(T - 1), jnp.float32)
                    for t in range(T - 1, _TLOW - 1, -1):
                        x = highbuf[t - _TLOW, pl.ds(goff, 16)]
                        acc2 = jnp.where(x > 0.5, jnp.float32(t), acc2)
                    lo = outbuf[pl.ds(goff, 16)]
                    outbuf[pl.ds(goff, 16)] = jnp.where(lo < _BIG, lo, acc2)
                    return carry2

                lax.fori_loop(0, ngrp, grp2, 0)

            pltpu.async_copy(
                outbuf, out_hbm.at[pl.ds(c0, _CHUNK)], sem
            ).wait()

        lax.fori_loop(0, rounds, round_body, 0)

    return pl.kernel(
        body,
        mesh=mesh,
        out_type=jax.ShapeDtypeStruct((BN,), jnp.float32),
        scratch_types=[
            pltpu.VMEM((_TLOW, _CHUNK), jnp.float32),
            pltpu.VMEM((thigh, _CHUNK), jnp.float32),
            pltpu.VMEM((_CHUNK,), jnp.float32),
            pltpu.VMEM((16,), jnp.int32),
            pltpu.SemaphoreType.DMA,
        ],
    )


def _tgt_krnl(tgt_ref, out_ref, *, T):
    tg = tgt_ref[...]
    out_ref[...] = jnp.where(tg < 0, tg + T, tg)


def kernel(spk_out, targets):
    T, B, N = spk_out.shape
    BN = B * N
    spk2 = spk_out.reshape(T, BN)

    first_flat = _sc_first_spike(T, BN)(spk2)

    import functools

    tgt_out = pl.pallas_call(
        functools.partial(_tgt_krnl, T=T),
        out_shape=jax.ShapeDtypeStruct((B, N), jnp.float32),
    )(targets)

    return first_flat.reshape(B, N), tgt_out


# speedup vs baseline: 1.6503x; 1.6503x over previous
import functools

import jax
import jax.numpy as jnp
from jax import lax
from jax.experimental import pallas as pl
from jax.experimental.pallas import tpu as pltpu
from jax.experimental.pallas import tpu_sc as plsc

_NW = 32
_BPW = 8
_TS = 8
_BIG = 1.0e9


def _goffs(N):
    offs = list(range(0, N - 15, 16))
    if offs[-1] + 16 < N:
        offs.append(N - 16)
    return offs


def _sc_first_spike(T, B, N):
    nstage = T // _TS
    ngrp = len(_goffs(N))
    tail = N % 16 != 0
    mesh = plsc.VectorSubcoreMesh(core_axis_name="c", subcore_axis_name="s")

    ngrp_main = ngrp - 1 if tail else ngrp
    tail_off = N - 16

    def goff_of(g):
        return pl.multiple_of(g * 16, 16)

    def body(x_hbm, out_hbm, buf, outbuf, flag, sem):
        cid = lax.axis_index("c")
        sid = lax.axis_index("s")
        wid = sid * 2 + cid
        b0 = pl.multiple_of(wid * _BPW, 8)

        for bb in range(_BPW):
            def init_grp(g, c, bb=bb):
                outbuf[bb, pl.ds(goff_of(g), 16)] = jnp.full(
                    (16,), _BIG, jnp.float32
                )
                return c

            lax.fori_loop(0, ngrp_main, init_grp, 0)
            if tail:
                outbuf[bb, pl.ds(tail_off, 16)] = jnp.full(
                    (16,), _BIG, jnp.float32
                )

        flag[0] = jnp.int32(0)

        def stage_step(stage, carry):
            @pl.when(flag[0] == 0)
            def _do_stage():
                t_base = (stage * _TS).astype(jnp.float32)
                pltpu.async_copy(
                    x_hbm.at[pl.ds(stage * _TS, _TS), pl.ds(b0, _BPW), :],
                    buf,
                    sem,
                ).wait()
                sil = jnp.zeros((16,), jnp.int32)
                for bb in range(_BPW):
                    def up_goff(goff, s, bb=bb, t_base=t_base):
                        acc = jnp.full((16,), _BIG, jnp.float32)
                        for t in range(_TS - 1, -1, -1):
                            x = buf[t, bb, pl.ds(goff, 16)]
                            acc = jnp.where(
                                x > 0.5, jnp.float32(t) + t_base, acc
                            )
                        old = outbuf[bb, pl.ds(goff, 16)]
                        new = jnp.where(old < _BIG, old, acc)
                        outbuf[bb, pl.ds(goff, 16)] = new
                        return s | jnp.where(new >= _BIG, 1, 0)

                    sil = lax.fori_loop(
                        0, ngrp_main,
                        lambda g, s, f=up_goff: f(goff_of(g), s), sil
                    )
                    if tail:
                        sil = up_goff(tail_off, sil)
                any_s = sil[0]
                for lane in range(1, 16):
                    any_s = any_s | sil[lane]
                flag[0] = jnp.where(any_s > 0, 0, 1)
            return carry

        lax.fori_loop(0, nstage, stage_step, 0)

        for bb in range(_BPW):
            def fin_goff(goff, bb=bb):
                v = outbuf[bb, pl.ds(goff, 16)]
                outbuf[bb, pl.ds(goff, 16)] = jnp.minimum(
                    v, jnp.float32(T - 1)
                )

            def fin_grp(g, c):
                fin_goff(goff_of(g))
                return c

            lax.fori_loop(0, ngrp_main, fin_grp, 0)
            if tail:
                fin_goff(tail_off)

        pltpu.async_copy(
            outbuf, out_hbm.at[pl.ds(b0, _BPW), :], sem
        ).wait()

    return pl.kernel(
        body,
        mesh=mesh,
        out_type=jax.ShapeDtypeStruct((B, N), jnp.float32),
        scratch_types=[
            pltpu.VMEM((_TS, _BPW, N), jnp.float32),
            pltpu.VMEM((_BPW, N), jnp.float32),
            pltpu.SMEM((1,), jnp.int32),
            pltpu.SemaphoreType.DMA,
        ],
    )


def _tgt_krnl(tgt_ref, out_ref, *, T):
    tg = tgt_ref[...]
    out_ref[...] = jnp.where(tg < 0, tg + T, tg)


def kernel(spk_out, targets):
    T, B, N = spk_out.shape

    first = _sc_first_spike(T, B, N)(spk_out)

    tgt_out = pl.pallas_call(
        functools.partial(_tgt_krnl, T=T),
        out_shape=jax.ShapeDtypeStruct((B, N), jnp.float32),
    )(targets)

    return first, tgt_out
